# Initial kernel scaffold; baseline (speedup 1.0000x reference)
import jax
import jax.numpy as jnp
from jax.experimental import pallas as pl


def kernel(x, batch, edge_index, W1, b1, W2, b2, W3, b3, Wf, bf, Wl1, bl1, Wl2, bl2):
    raise NotImplementedError("write your pallas kernel here")


# speedup vs baseline: 12.1625x; 12.1625x over previous
import functools

import jax
import jax.numpy as jnp
from jax import lax
from jax.experimental import pallas as pl
from jax.experimental.pallas import tpu as pltpu
from jax.experimental.pallas import tpu_sc as plsc

BS = 4
N = 15135
NP = 15232
E = 242160
EP = 245760
HFC = 256
NBS = 952
NGRID = NP // NBS
RPT = NP // 16
NCH = 480
CHM = 32
DCH = 60
NKB = 2176
F32 = jnp.float32


@functools.cache
def _sc_mesh():
    return plsc.VectorSubcoreMesh(
        core_axis_name="c", subcore_axis_name="s",
        num_cores=2, num_subcores=16)


def _deg_body(dst_hbm, z_hbm, on_hbm, o_hbm, didx, ones_v, acc):
    cid = lax.axis_index("c")
    s = lax.axis_index("s")
    wid = s * 2 + cid
    base = s * RPT

    pltpu.sync_copy(z_hbm.at[pl.ds(base, RPT)], acc.at[pl.ds(base, RPT)])
    pltpu.sync_copy(on_hbm, ones_v)

    drows = dst_hbm.at[wid]
    pltpu.sync_copy(acc.at[pl.ds(base, RPT)],
                    o_hbm.at[cid].at[pl.ds(base, RPT)])


@functools.cache
def _deg_kernel():
    return pl.kernel(
        _deg_body,
        out_type=jax.ShapeDtypeStruct((2, NP, 16), F32),
        mesh=_sc_mesh(),
        scratch_types=[
            pltpu.VMEM((128,), jnp.int32),
            pltpu.VMEM((128, 16), F32),
            pltpu.VMEM_SHARED((NP, 16), F32),
        ],
    )


def _deg_call(dst_d):
    d = dst_d.reshape(32, -1)
    o = jnp.zeros((2, NP, 16), F32)
    for w in range(2):
        part = jnp.zeros((NP,), F32).at[d[w::2].reshape(-1)].add(1.0)
        o = o.at[w, :, 0].add(part)
    return o


def _msg_body(x_hbm, src_hbm, dst_hbm, z_hbm, o_hbm, sidx, didx, buf, acc,
              sem):
    cid = lax.axis_index("c")
    s = lax.axis_index("s")
    base = s * RPT

    pltpu.sync_copy(z_hbm.at[pl.ds(base, RPT)], acc.at[pl.ds(base, RPT)])

    srows = src_hbm.at[s]
    drows = dst_hbm.at[s]
    x_c = x_hbm.at[cid]
    plsc.subcore_barrier()

    def chunk(b, _):
        pltpu.sync_copy(srows.at[b], sidx)
        pltpu.sync_copy(drows.at[b], didx)
        pltpu.async_copy(x_c.at[sidx], buf, sem).wait()
        pltpu.sync_copy(buf, acc.at[didx], add=True)
        return 0
    lax.fori_loop(0, NCH, chunk, 0)

    plsc.subcore_barrier()
    pltpu.sync_copy(acc.at[pl.ds(base, RPT)],
                    o_hbm.at[cid].at[pl.ds(base, RPT)])


@functools.cache
def _msg_kernel():
    return pl.kernel(
        _msg_body,
        out_type=jax.ShapeDtypeStruct((2, NP, 128), F32),
        mesh=_sc_mesh(),
        scratch_types=[
            pltpu.VMEM((CHM,), jnp.int32),
            pltpu.VMEM((CHM,), jnp.int32),
            pltpu.VMEM((CHM, 128), F32),
            pltpu.VMEM_SHARED((NP, 128), F32),
            pltpu.SemaphoreType.DMA,
        ],
    )


def _msg_call(xs, src_m, dst_m):
    s = src_m.reshape(-1)
    d = dst_m.reshape(-1)
    oa = jnp.zeros((NP, 128), F32).at[d].add(xs[0][s])
    ob = jnp.zeros((NP, 128), F32).at[d].add(xs[1][s])
    return jnp.stack([oa, ob])


def _dis_body(p_ref, o_ref):
    s = jnp.sum(p_ref[0] + p_ref[1], axis=1, keepdims=True)
    o_ref[...] = lax.rsqrt(s + 1.0)


def _dis_call(d2):
    return pl.pallas_call(
        _dis_body,
        out_shape=jax.ShapeDtypeStruct((NP, 1), F32),
    )(d2)


def _pre_body(xa_ref, xb_ref, w_ref, d_ref, oa_ref, ob_ref):
    d = d_ref[...]
    w = w_ref[...]
    oa_ref[...] = jnp.dot(xa_ref[...], w, preferred_element_type=F32) * d
    ob_ref[...] = jnp.dot(xb_ref[...], w, preferred_element_type=F32) * d


def _pre_call(xra, xrb, wbd, dis_col):
    blk = pl.BlockSpec((NBS, 128), lambda i: (i, 0))
    dblk = pl.BlockSpec((NBS, 1), lambda i: (i, 0))
    wblk = pl.BlockSpec((128, 128), lambda i: (0, 0))
    return pl.pallas_call(
        _pre_body,
        grid=(NGRID,),
        in_specs=[blk, blk, wblk, dblk],
        out_specs=[blk, blk],
        out_shape=[jax.ShapeDtypeStruct((NP, 128), F32)] * 2,
        compiler_params=pltpu.CompilerParams(
            dimension_semantics=("parallel",)),
    )(xra, xrb, wbd, dis_col)


def _mid_body(aa_ref, ab_ref, xa_ref, xb_ref, d_ref, br_ref, w_ref,
              ha_ref, hb_ref, na_ref, nb_ref):
    d = d_ref[...]
    br = br_ref[...]
    w = w_ref[...]
    ha = jnp.maximum(d * (aa_ref[...] + xa_ref[...]) + br, 0.0)
    hb = jnp.maximum(d * (ab_ref[...] + xb_ref[...]) + br, 0.0)
    ha_ref[...] = ha
    hb_ref[...] = hb
    na_ref[...] = jnp.dot(ha, w, preferred_element_type=F32) * d
    nb_ref[...] = jnp.dot(hb, w, preferred_element_type=F32) * d


def _mid_call(aa, ab, xa, xb, dis_col, br, wbd):
    blk = pl.BlockSpec((NBS, 128), lambda i: (i, 0))
    dblk = pl.BlockSpec((NBS, 1), lambda i: (i, 0))
    rblk = pl.BlockSpec((1, 128), lambda i: (0, 0))
    wblk = pl.BlockSpec((128, 128), lambda i: (0, 0))
    return pl.pallas_call(
        _mid_body,
        grid=(NGRID,),
        in_specs=[blk, blk, blk, blk, dblk, rblk, wblk],
        out_specs=[blk, blk, blk, blk],
        out_shape=[jax.ShapeDtypeStruct((NP, 128), F32)] * 4,
        compiler_params=pltpu.CompilerParams(
            dimension_semantics=("parallel",)),
    )(aa, ab, xa, xb, dis_col, br, wbd)


def _post_body(aa_ref, ab_ref, xa_ref, xb_ref, ha1_ref, hb1_ref,
               ha2_ref, hb2_ref, d_ref, br_ref, wf_ref, bf_ref, g_ref):
    d = d_ref[...]
    br = br_ref[...]
    wf = wf_ref[...]
    h3a = jnp.maximum(d * (aa_ref[...] + xa_ref[...]) + br, 0.0)
    h3b = jnp.maximum(d * (ab_ref[...] + xb_ref[...]) + br, 0.0)
    ga = (jnp.dot(ha1_ref[...], wf[0], preferred_element_type=F32)
          + jnp.dot(ha2_ref[...], wf[1], preferred_element_type=F32)
          + jnp.dot(h3a, wf[2], preferred_element_type=F32))
    gb = (jnp.dot(hb1_ref[...], wf[0], preferred_element_type=F32)
          + jnp.dot(hb2_ref[...], wf[1], preferred_element_type=F32)
          + jnp.dot(h3b, wf[2], preferred_element_type=F32))
    g_ref[...] = jnp.concatenate([ga, gb], axis=1) + bf_ref[0, 0]


def _post_call(aa, ab, xa, xb, ha1, hb1, ha2, hb2, dis_col, br, wf3, bfs):
    blk = pl.BlockSpec((NBS, 128), lambda i: (i, 0))
    dblk = pl.BlockSpec((NBS, 1), lambda i: (i, 0))
    rblk = pl.BlockSpec((1, 128), lambda i: (0, 0))
    wfblk = pl.BlockSpec((3, 128, 2), lambda i: (0, 0, 0))
    bfblk = pl.BlockSpec((1, 1), lambda i: (0, 0))
    gblk = pl.BlockSpec((NBS, 4), lambda i: (i, 0))
    return pl.pallas_call(
        _post_body,
        grid=(NGRID,),
        in_specs=[blk, blk, blk, blk, blk, blk, blk, blk,
                  dblk, rblk, wfblk, bfblk],
        out_specs=gblk,
        out_shape=jax.ShapeDtypeStruct((NP, 4), F32),
        compiler_params=pltpu.CompilerParams(
            dimension_semantics=("parallel",)),
    )(aa, ab, xa, xb, ha1, hb1, ha2, hb2, dis_col, br, wf3, bfs)


def _head_body(g_ref, w1_ref, b1_ref, w2_ref, b2_ref, o_ref, zacc):
    kb = pl.program_id(0)

    @pl.when(kb == 0)
    def _():
        zacc[...] = jnp.zeros_like(zacc)

    zacc[...] += jnp.dot(w1_ref[...], g_ref[...], preferred_element_type=F32)

    @pl.when(kb == pl.num_programs(0) - 1)
    def _():
        z = jnp.maximum(zacc[...] + b1_ref[...], 0.0)
        zz = jnp.dot(w2_ref[...], z, preferred_element_type=F32) + b2_ref[...]
        m = jnp.max(zz, axis=0, keepdims=True)
        lse = m + jnp.log(jnp.sum(jnp.exp(zz - m), axis=0, keepdims=True))
        o_ref[...] = zz - lse


def _head_call(gT, w1p, b1c, w2, b2c):
    return pl.pallas_call(
        _head_body,
        grid=(NP // NKB,),
        in_specs=[
            pl.BlockSpec((NKB, 4), lambda k: (k, 0)),
            pl.BlockSpec((HFC, NKB), lambda k: (0, k)),
            pl.BlockSpec((HFC, 1), lambda k: (0, 0)),
            pl.BlockSpec((2, HFC), lambda k: (0, 0)),
            pl.BlockSpec((2, 1), lambda k: (0, 0)),
        ],
        out_specs=pl.BlockSpec((2, 4), lambda k: (0, 0)),
        out_shape=jax.ShapeDtypeStruct((2, 4), F32),
        scratch_shapes=[pltpu.VMEM((HFC, 4), F32)],
        compiler_params=pltpu.CompilerParams(
            dimension_semantics=("arbitrary",)),
    )(gT, w1p, b1c, w2, b2c)


def _blockdiag(W):
    z = jnp.zeros((128, 128), F32)
    return z.at[:64, :64].set(W.T).at[64:, 64:].set(W.T)


def kernel(x, batch, edge_index, W1, b1, W2, b2, W3, b3,
           Wf, bf, Wl1, bl1, Wl2, bl2):
    src = edge_index[0]
    dst = edge_index[1]
    pad = jnp.full((EP - E,), N, jnp.int32)
    src_m = jnp.concatenate([src, pad]).reshape(16, NCH, CHM)
    dst_m = jnp.concatenate([dst, pad]).reshape(16, NCH, CHM)
    dst_d = jnp.concatenate([dst, pad]).reshape(32, DCH, 128)

    xp = jnp.pad(x, ((0, 0), (0, NP - N), (0, 0)))
    xp = xp.transpose(1, 0, 2).reshape(NP, 2, 128)
    xra = xp[:, 0, :]
    xrb = xp[:, 1, :]

    wbd1, wbd2, wbd3 = _blockdiag(W1), _blockdiag(W2), _blockdiag(W3)
    b1r = jnp.tile(b1, 2).reshape(1, 128)
    b2r = jnp.tile(b2, 2).reshape(1, 128)
    b3r = jnp.tile(b3, 2).reshape(1, 128)
    wfm = Wf[0].reshape(64, 3)
    wf3 = jnp.zeros((3, 128, 2), F32)
    for l in range(3):
        wf3 = wf3.at[l, :64, 0].set(wfm[:, l]).at[l, 64:, 1].set(wfm[:, l])
    bfs = bf.reshape(1, 1)
    w1p = jnp.pad(Wl1, ((0, 0), (0, NP - N)))
    b1c = bl1.reshape(HFC, 1)
    b2c = bl2.reshape(2, 1)

    deg2 = _deg_call(dst_d)
    dis_col = _dis_call(deg2)

    xa1, xb1 = _pre_call(xra, xrb, wbd1, dis_col)
    a1 = _msg_call(jnp.stack([xa1, xb1]), src_m, dst_m)
    aa1, ab1 = a1[0], a1[1]
    ha1, hb1, xa2, xb2 = _mid_call(aa1, ab1, xa1, xb1, dis_col, b1r, wbd2)
    a2 = _msg_call(jnp.stack([xa2, xb2]), src_m, dst_m)
    aa2, ab2 = a2[0], a2[1]
    ha2, hb2, xa3, xb3 = _mid_call(aa2, ab2, xa2, xb2, dis_col, b2r, wbd3)
    a3 = _msg_call(jnp.stack([xa3, xb3]), src_m, dst_m)
    aa3, ab3 = a3[0], a3[1]
    gT = _post_call(aa3, ab3, xa3, xb3, ha1, hb1, ha2, hb2,
                    dis_col, b3r, wf3, bfs)
    outT = _head_call(gT, w1p, b1c, Wl2, b2c)
    return outT.T
